# baseline (speedup 1.0000x reference)
import jax
import jax.numpy as jnp
from jax import lax
from jax.experimental import pallas as pl
from jax.experimental.pallas import tpu as pltpu
from jax.experimental.pallas import tpu_sc as plsc

N_NODES = 10000
N_PAD = 10240
N_EDGES = 320000
D_IN = 128
D_OUT = 128

NC = 2
NS = 16
NW = NC * NS

EPW = N_EDGES // NW
CHUNK = 80
NCH = EPW // CHUNK
ROWS_PER_TILE = N_PAD // NS


NIQ = 8
NRB = 4


def _sc_aggregate_body(h_hbm, src_hbm, dst_hbm, ones_hbm, z128_hbm, z1_hbm,
                       out_s_hbm, out_d_hbm, *refs):
    sis = list(refs[0:NIQ])
    dis = list(refs[NIQ:2 * NIQ])
    rbs = list(refs[2 * NIQ:2 * NIQ + NRB])
    ones_v = refs[2 * NIQ + NRB]
    acc_sh = refs[2 * NIQ + NRB + 1]
    deg_sh = refs[2 * NIQ + NRB + 2]
    sems = refs[2 * NIQ + NRB + 3:]
    gis = list(sems[0:NIQ])
    gs = list(sems[NIQ:NIQ + NRB])
    ss = list(sems[NIQ + NRB:NIQ + 2 * NRB])
    dsem = sems[NIQ + 2 * NRB]
    cid = lax.axis_index("c")
    sid = lax.axis_index("s")
    wid = cid * NS + sid

    row0 = pl.multiple_of(sid * ROWS_PER_TILE, 8)
    rsl = pl.ds(row0, ROWS_PER_TILE)
    pltpu.sync_copy(z128_hbm, rbs[0])
    for k in range(ROWS_PER_TILE // CHUNK):
        pltpu.sync_copy(rbs[0], acc_sh.at[pl.ds(row0 + k * CHUNK, CHUNK)])
    pltpu.sync_copy(z1_hbm.at[rsl], deg_sh.at[rsl])
    pltpu.sync_copy(ones_hbm, ones_v)
    plsc.subcore_barrier()

    ebase = wid * EPW

    def esl(t):
        return pl.ds(pl.multiple_of(ebase + t * CHUNK, 8), CHUNK)

    def fire_idx(t, q):
        pltpu.async_copy(src_hbm.at[esl(t)], sis[q], gis[q])
        pltpu.async_copy(dst_hbm.at[esl(t)], dis[q], gis[q])

    def wait_idx(q):
        pltpu.make_async_copy(src_hbm.at[pl.ds(0, CHUNK)], sis[q],
                              gis[q]).wait()
        pltpu.make_async_copy(src_hbm.at[pl.ds(0, CHUNK)], dis[q],
                              gis[q]).wait()

    def fire_gather(q, u):
        pltpu.async_copy(h_hbm.at[sis[q]], rbs[u], gs[u])

    def wait_gather(q, u):
        pltpu.make_async_copy(h_hbm.at[sis[q]], rbs[u], gs[u]).wait()

    def fire_scatter(q, u):
        pltpu.async_copy(ones_v, deg_sh.at[dis[q]], dsem, add=True)
        pltpu.async_copy(rbs[u], acc_sh.at[dis[q]], ss[u], add=True)

    def wait_scatter(q, u):
        pltpu.make_async_copy(rbs[u], acc_sh.at[dis[q]], ss[u]).wait()
        pltpu.make_async_copy(ones_v, deg_sh.at[dis[q]], dsem).wait()

    for q in range(6):
        fire_idx(q, q)
    for u in range(2):
        wait_idx(u)
        fire_gather(u, u)

    UNROLL = 8
    MAIN = (NCH - 5) // UNROLL * UNROLL

    def outer(k, carry):
        for j in range(UNROLL):
            r = j
            u = j % NRB
            rm2 = (j - 2) % NIQ
            um2 = (j - 2) % NRB
            t = k * UNROLL + j

            @pl.when(t >= 2)
            def _():
                wait_scatter(rm2, um2)

            @pl.when(t + 6 < NCH)
            def _():
                fire_idx(t + 6, (j + 6) % NIQ)

            wait_gather(r, u)
            fire_scatter(r, u)

            @pl.when(t + 2 < NCH)
            def _():
                wait_idx((j + 2) % NIQ)
                fire_gather((j + 2) % NIQ, (j + 2) % NRB)
        return carry

    lax.fori_loop(0, MAIN // UNROLL, outer, 0)

    for t in range(MAIN, NCH):
        wait_scatter((t - 2) % NIQ, (t - 2) % NRB)
        if t + 6 < NCH:
            fire_idx(t + 6, (t + 6) % NIQ)
        wait_gather(t % NIQ, t % NRB)
        fire_scatter(t % NIQ, t % NRB)
        if t + 2 < NCH:
            wait_idx((t + 2) % NIQ)
            fire_gather((t + 2) % NIQ, (t + 2) % NRB)
    for t in range(NCH - 2, NCH):
        wait_scatter(t % NIQ, t % NRB)

    plsc.subcore_barrier()
    pltpu.sync_copy(acc_sh.at[rsl], out_s_hbm.at[cid, rsl])
    pltpu.sync_copy(deg_sh.at[rsl], out_d_hbm.at[cid, rsl])


@jax.jit
def _sc_aggregate(h, src, dst, ones, z128, z1):
    mesh = plsc.VectorSubcoreMesh(core_axis_name="c", subcore_axis_name="s")
    return pl.kernel(
        _sc_aggregate_body,
        out_type=[
            jax.ShapeDtypeStruct((NC, N_PAD, D_IN), jnp.float32),
            jax.ShapeDtypeStruct((NC, N_PAD), jnp.float32),
        ],
        mesh=mesh,
        scratch_types=(
            [pltpu.VMEM((CHUNK,), jnp.int32)] * (2 * NIQ)
            + [pltpu.VMEM((CHUNK, D_IN), jnp.float32)] * NRB
            + [
                pltpu.VMEM((CHUNK,), jnp.float32),
                pltpu.VMEM_SHARED((N_PAD, D_IN), jnp.float32),
                pltpu.VMEM_SHARED((N_PAD,), jnp.float32),
            ]
            + [pltpu.SemaphoreType.DMA] * (NIQ + 2 * NRB + 1)
        ),
    )(h, src, dst, ones, z128, z1)


_TC_R = 1024


def _tc_part1_body(h_ref, wt1_ref, b_ref, out_ref):
    out_ref[...] = (
        jnp.dot(h_ref[...], wt1_ref[...], preferred_element_type=jnp.float32)
        + b_ref[...]
    )


@jax.jit
def _tc_part1(h, wt1, b2d):
    r = 1000
    return pl.pallas_call(
        _tc_part1_body,
        grid=(N_NODES // r,),
        in_specs=[
            pl.BlockSpec((r, D_IN), lambda i: (i, 0)),
            pl.BlockSpec((D_IN, D_OUT), lambda i: (0, 0)),
            pl.BlockSpec((1, D_OUT), lambda i: (0, 0)),
        ],
        out_specs=pl.BlockSpec((r, D_OUT), lambda i: (i, 0)),
        out_shape=jax.ShapeDtypeStruct((N_NODES, D_OUT), jnp.float32),
    )(h, wt1, b2d)


def _tc_combine_body(p1_ref, s_ref, d_ref, wt_ref, out_ref):
    s = s_ref[0] + s_ref[1]
    dg = d_ref[0] + d_ref[1]
    rb = _TC_R // 128
    ri = lax.broadcasted_iota(jnp.int32, (_TC_R, rb), 0) // 128
    ci = lax.broadcasted_iota(jnp.int32, (_TC_R, rb), 1)
    sel = (ri == ci).astype(jnp.float32)
    t = jnp.dot(sel, dg, preferred_element_type=jnp.float32)
    ji = lax.broadcasted_iota(jnp.int32, (_TC_R, 128), 0) % 128
    ki = lax.broadcasted_iota(jnp.int32, (_TC_R, 128), 1)
    deg_col = jnp.sum(jnp.where(ji == ki, t, 0.0), axis=1, keepdims=True)
    h_n = s / jnp.maximum(deg_col, 1.0)
    out_ref[...] = (
        p1_ref[...]
        + jnp.dot(h_n, wt_ref[...], preferred_element_type=jnp.float32)
    )


@jax.jit
def _tc_combine(p1pad, s_parts, d_parts, wt2):
    grid = (N_PAD // _TC_R,)
    rb = _TC_R // 128
    return pl.pallas_call(
        _tc_combine_body,
        grid=grid,
        in_specs=[
            pl.BlockSpec((_TC_R, D_OUT), lambda i: (i, 0)),
            pl.BlockSpec((NC, _TC_R, D_IN), lambda i: (0, i, 0)),
            pl.BlockSpec((NC, rb, 128), lambda i: (0, i, 0)),
            pl.BlockSpec((D_IN, D_OUT), lambda i: (0, 0)),
        ],
        out_specs=pl.BlockSpec((_TC_R, D_OUT), lambda i: (i, 0)),
        out_shape=jax.ShapeDtypeStruct((N_PAD, D_OUT), jnp.float32),
    )(p1pad, s_parts, d_parts, wt2)


def kernel(h, edge_index, W, b):
    ei = edge_index.astype(jnp.int32)
    src = ei[0]
    dst = ei[1]
    ones = jnp.ones((CHUNK,), jnp.float32)
    z128 = jnp.zeros((CHUNK, D_IN), jnp.float32)
    z1 = jnp.zeros((N_PAD,), jnp.float32)
    wt = W.T
    b2d = b.reshape(1, D_OUT)
    s_parts, d_parts = _sc_aggregate(h, src, dst, ones, z128, z1)
    p1 = _tc_part1(h, wt[:D_IN], b2d)
    p1pad = jnp.pad(p1, ((0, N_PAD - N_NODES), (0, 0)))
    d_parts = d_parts.reshape(NC, N_PAD // 128, 128)
    out = _tc_combine(p1pad, s_parts, d_parts, wt[D_IN:])
    return out[:N_NODES]


# speedup vs baseline: 1.0189x; 1.0189x over previous
import jax
import jax.numpy as jnp
from jax import lax
from jax.experimental import pallas as pl
from jax.experimental.pallas import tpu as pltpu
from jax.experimental.pallas import tpu_sc as plsc

N_NODES = 10000
N_PAD = 10240
N_EDGES = 320000
D_IN = 128
D_OUT = 128

NC = 2
NS = 16
NW = NC * NS

EPW = N_EDGES // NW
CHUNK = 80
NCH = EPW // CHUNK
ROWS_PER_TILE = N_PAD // NS


NIQ = 8
NRB = 4


def _sc_aggregate_body(h_hbm, src_hbm, dst_hbm, ones_hbm, z128_hbm, z1_hbm,
                       out_s_hbm, out_d_hbm, *refs):
    sis = list(refs[0:NIQ])
    dis = list(refs[NIQ:2 * NIQ])
    rbs = list(refs[2 * NIQ:2 * NIQ + NRB])
    ones_v = refs[2 * NIQ + NRB]
    acc_sh = refs[2 * NIQ + NRB + 1]
    deg_sh = refs[2 * NIQ + NRB + 2]
    sems = refs[2 * NIQ + NRB + 3:]
    gis = list(sems[0:NIQ])
    gs = list(sems[NIQ:NIQ + NRB])
    ss = list(sems[NIQ + NRB:NIQ + 2 * NRB])
    dsem = sems[NIQ + 2 * NRB]
    cid = lax.axis_index("c")
    sid = lax.axis_index("s")
    wid = cid * NS + sid

    row0 = pl.multiple_of(sid * ROWS_PER_TILE, 8)
    rsl = pl.ds(row0, ROWS_PER_TILE)
    pltpu.sync_copy(z128_hbm, rbs[0])
    for k in range(ROWS_PER_TILE // CHUNK):
        pltpu.sync_copy(rbs[0], acc_sh.at[pl.ds(row0 + k * CHUNK, CHUNK)])
    pltpu.sync_copy(z1_hbm.at[rsl], deg_sh.at[rsl])
    pltpu.sync_copy(ones_hbm, ones_v)
    plsc.subcore_barrier()

    ebase = wid * EPW

    def esl(t):
        return pl.ds(pl.multiple_of(ebase + t * CHUNK, 8), CHUNK)

    def fire_idx(t, q):
        pltpu.async_copy(src_hbm.at[esl(t)], sis[q], gis[q])
        pltpu.async_copy(dst_hbm.at[esl(t)], dis[q], gis[q])

    def wait_idx(q):
        pltpu.make_async_copy(src_hbm.at[pl.ds(0, CHUNK)], sis[q],
                              gis[q]).wait()
        pltpu.make_async_copy(src_hbm.at[pl.ds(0, CHUNK)], dis[q],
                              gis[q]).wait()

    def fire_gather(q, u):
        pltpu.async_copy(h_hbm.at[sis[q]], rbs[u], gs[u])

    def wait_gather(q, u):
        pltpu.make_async_copy(h_hbm.at[sis[q]], rbs[u], gs[u]).wait()

    def fire_scatter(q, u):
        pltpu.async_copy(ones_v, deg_sh.at[dis[q]], dsem, add=True)
        pltpu.async_copy(rbs[u], acc_sh.at[dis[q]], ss[u], add=True)

    def wait_scatter(q, u):
        pltpu.make_async_copy(rbs[u], acc_sh.at[dis[q]], ss[u]).wait()
        pltpu.make_async_copy(ones_v, deg_sh.at[dis[q]], dsem).wait()

    for q in range(6):
        fire_idx(q, q)
    for u in range(2):
        wait_idx(u)
        fire_gather(u, u)

    UNROLL = 8
    MAIN = (NCH - 5) // UNROLL * UNROLL

    def outer(k, carry):
        for j in range(UNROLL):
            r = j
            u = j % NRB
            rm2 = (j - 2) % NIQ
            um2 = (j - 2) % NRB
            t = k * UNROLL + j

            @pl.when(t >= 2)
            def _():
                wait_scatter(rm2, um2)

            @pl.when(t + 6 < NCH)
            def _():
                fire_idx(t + 6, (j + 6) % NIQ)

            wait_gather(r, u)
            fire_scatter(r, u)

            @pl.when(t + 2 < NCH)
            def _():
                wait_idx((j + 2) % NIQ)
                fire_gather((j + 2) % NIQ, (j + 2) % NRB)
        return carry

    lax.fori_loop(0, MAIN // UNROLL, outer, 0)

    for t in range(MAIN, NCH):
        wait_scatter((t - 2) % NIQ, (t - 2) % NRB)
        if t + 6 < NCH:
            fire_idx(t + 6, (t + 6) % NIQ)
        wait_gather(t % NIQ, t % NRB)
        fire_scatter(t % NIQ, t % NRB)
        if t + 2 < NCH:
            wait_idx((t + 2) % NIQ)
            fire_gather((t + 2) % NIQ, (t + 2) % NRB)
    for t in range(NCH - 2, NCH):
        wait_scatter(t % NIQ, t % NRB)

    plsc.subcore_barrier()
    for k in range(ROWS_PER_TILE // CHUNK):
        u = k % 2
        ksl = pl.ds(row0 + k * CHUNK, CHUNK)
        if k >= 2:
            pltpu.make_async_copy(rbs[u], out_s_hbm.at[cid, ksl],
                                  ss[u]).wait()
        pltpu.sync_copy(acc_sh.at[ksl], rbs[u])
        pltpu.async_copy(rbs[u], out_s_hbm.at[cid, ksl], ss[u])
    for u in range(2):
        pltpu.make_async_copy(rbs[u], out_s_hbm.at[cid, pl.ds(row0, CHUNK)],
                              ss[u]).wait()
    pltpu.sync_copy(deg_sh.at[rsl], out_d_hbm.at[cid, rsl])


@jax.jit
def _sc_aggregate(h, src, dst, ones, z128, z1):
    mesh = plsc.VectorSubcoreMesh(core_axis_name="c", subcore_axis_name="s")
    return pl.kernel(
        _sc_aggregate_body,
        out_type=[
            jax.ShapeDtypeStruct((NC, N_PAD, D_IN), jnp.float32),
            jax.ShapeDtypeStruct((NC, N_PAD), jnp.float32),
        ],
        mesh=mesh,
        scratch_types=(
            [pltpu.VMEM((CHUNK,), jnp.int32)] * (2 * NIQ)
            + [pltpu.VMEM((CHUNK, D_IN), jnp.float32)] * NRB
            + [
                pltpu.VMEM((CHUNK,), jnp.float32),
                pltpu.VMEM_SHARED((N_PAD, D_IN), jnp.float32),
                pltpu.VMEM_SHARED((N_PAD,), jnp.float32),
            ]
            + [pltpu.SemaphoreType.DMA] * (NIQ + 2 * NRB + 1)
        ),
    )(h, src, dst, ones, z128, z1)


_TC_R = 1024


def _tc_part1_body(h_ref, wt1_ref, b_ref, out_ref):
    out_ref[...] = (
        jnp.dot(h_ref[...], wt1_ref[...], preferred_element_type=jnp.float32)
        + b_ref[...]
    )


@jax.jit
def _tc_part1(h, wt1, b2d):
    return pl.pallas_call(
        _tc_part1_body,
        grid=(N_PAD // _TC_R,),
        in_specs=[
            pl.BlockSpec((_TC_R, D_IN), lambda i: (i, 0)),
            pl.BlockSpec((D_IN, D_OUT), lambda i: (0, 0)),
            pl.BlockSpec((1, D_OUT), lambda i: (0, 0)),
        ],
        out_specs=pl.BlockSpec((_TC_R, D_OUT), lambda i: (i, 0)),
        out_shape=jax.ShapeDtypeStruct((N_PAD, D_OUT), jnp.float32),
    )(h, wt1, b2d)


def _tc_combine_body(p1_ref, s_ref, d_ref, wt_ref, out_ref):
    s = s_ref[0] + s_ref[1]
    dg = d_ref[0] + d_ref[1]
    rb = _TC_R // 128
    ri = lax.broadcasted_iota(jnp.int32, (_TC_R, rb), 0) // 128
    ci = lax.broadcasted_iota(jnp.int32, (_TC_R, rb), 1)
    sel = (ri == ci).astype(jnp.float32)
    t = jnp.dot(sel, dg, preferred_element_type=jnp.float32)
    ji = lax.broadcasted_iota(jnp.int32, (_TC_R, 128), 0) % 128
    ki = lax.broadcasted_iota(jnp.int32, (_TC_R, 128), 1)
    deg_col = jnp.sum(jnp.where(ji == ki, t, 0.0), axis=1, keepdims=True)
    h_n = s / jnp.maximum(deg_col, 1.0)
    out_ref[...] = (
        p1_ref[...]
        + jnp.dot(h_n, wt_ref[...], preferred_element_type=jnp.float32)
    )


@jax.jit
def _tc_combine(p1pad, s_parts, d_parts, wt2):
    grid = (N_PAD // _TC_R,)
    rb = _TC_R // 128
    return pl.pallas_call(
        _tc_combine_body,
        grid=grid,
        in_specs=[
            pl.BlockSpec((_TC_R, D_OUT), lambda i: (i, 0)),
            pl.BlockSpec((NC, _TC_R, D_IN), lambda i: (0, i, 0)),
            pl.BlockSpec((NC, rb, 128), lambda i: (0, i, 0)),
            pl.BlockSpec((D_IN, D_OUT), lambda i: (0, 0)),
        ],
        out_specs=pl.BlockSpec((_TC_R, D_OUT), lambda i: (i, 0)),
        out_shape=jax.ShapeDtypeStruct((N_PAD, D_OUT), jnp.float32),
    )(p1pad, s_parts, d_parts, wt2)


def kernel(h, edge_index, W, b):
    ei = edge_index.astype(jnp.int32)
    src = ei[0]
    dst = ei[1]
    ones = jnp.ones((CHUNK,), jnp.float32)
    z128 = jnp.zeros((CHUNK, D_IN), jnp.float32)
    z1 = jnp.zeros((N_PAD,), jnp.float32)
    wt = W.T
    b2d = b.reshape(1, D_OUT)
    s_parts, d_parts = _sc_aggregate(h, src, dst, ones, z128, z1)
    p1pad = _tc_part1(h, wt[:D_IN], b2d)
    d_parts = d_parts.reshape(NC, N_PAD // 128, 128)
    out = _tc_combine(p1pad, s_parts, d_parts, wt[D_IN:])
    return out[:N_NODES]


# speedup vs baseline: 1.0468x; 1.0274x over previous
import jax
import jax.numpy as jnp
from jax import lax
from jax.experimental import pallas as pl
from jax.experimental.pallas import tpu as pltpu
from jax.experimental.pallas import tpu_sc as plsc

N_NODES = 10000
N_PAD = 10240
N_EDGES = 320000
D_IN = 128
D_OUT = 128

NC = 2
NS = 16
NW = NC * NS

EPW = N_EDGES // NW
CHUNK = 80
NCH = EPW // CHUNK
ROWS_PER_TILE = N_PAD // NS


NIQ = 8
NRB = 4


def _sc_aggregate_body(h_hbm, src_hbm, dst_hbm, ones_hbm, z128_hbm, z1_hbm,
                       out_s_hbm, out_d_hbm, *refs):
    sis = list(refs[0:NIQ])
    dis = list(refs[NIQ:2 * NIQ])
    rbs = list(refs[2 * NIQ:2 * NIQ + NRB])
    ones_v = refs[2 * NIQ + NRB]
    acc_sh = refs[2 * NIQ + NRB + 1]
    deg_sh = refs[2 * NIQ + NRB + 2]
    sems = refs[2 * NIQ + NRB + 3:]
    gis = list(sems[0:NIQ])
    gs = list(sems[NIQ:NIQ + NRB])
    ss = list(sems[NIQ + NRB:NIQ + 2 * NRB])
    dsem = sems[NIQ + 2 * NRB]
    cid = lax.axis_index("c")
    sid = lax.axis_index("s")
    wid = cid * NS + sid

    row0 = pl.multiple_of(sid * ROWS_PER_TILE, 8)
    rsl = pl.ds(row0, ROWS_PER_TILE)
    ebase = wid * EPW

    def esl(t):
        return pl.ds(pl.multiple_of(ebase + t * CHUNK, 8), CHUNK)

    def fire_idx(t, q):
        pltpu.async_copy(src_hbm.at[esl(t)], sis[q], gis[q])
        pltpu.async_copy(dst_hbm.at[esl(t)], dis[q], gis[q])

    def wait_idx(q):
        pltpu.make_async_copy(src_hbm.at[pl.ds(0, CHUNK)], sis[q],
                              gis[q]).wait()
        pltpu.make_async_copy(src_hbm.at[pl.ds(0, CHUNK)], dis[q],
                              gis[q]).wait()

    def fire_gather(q, u):
        pltpu.async_copy(h_hbm.at[sis[q]], rbs[u], gs[u])

    def wait_gather(q, u):
        pltpu.make_async_copy(h_hbm.at[sis[q]], rbs[u], gs[u]).wait()

    def fire_scatter(q, u):
        pltpu.async_copy(ones_v, deg_sh.at[dis[q]], dsem, add=True)
        pltpu.async_copy(rbs[u], acc_sh.at[dis[q]], ss[u], add=True)

    def wait_scatter(q, u):
        pltpu.make_async_copy(rbs[u], acc_sh.at[dis[q]], ss[u]).wait()
        pltpu.make_async_copy(ones_v, deg_sh.at[dis[q]], dsem).wait()

    for q in range(6):
        fire_idx(q, q)
    pltpu.sync_copy(z128_hbm, rbs[0])
    for k in range(ROWS_PER_TILE // CHUNK):
        pltpu.async_copy(rbs[0], acc_sh.at[pl.ds(row0 + k * CHUNK, CHUNK)],
                         ss[k % NRB])
    pltpu.sync_copy(z1_hbm.at[rsl], deg_sh.at[rsl])
    pltpu.sync_copy(ones_hbm, ones_v)
    for k in range(ROWS_PER_TILE // CHUNK):
        pltpu.make_async_copy(rbs[0],
                              acc_sh.at[pl.ds(row0 + k * CHUNK, CHUNK)],
                              ss[k % NRB]).wait()
    for u in range(2):
        wait_idx(u)
        fire_gather(u, u)
    plsc.subcore_barrier()

    UNROLL = 8
    MAIN = (NCH - 5) // UNROLL * UNROLL

    def outer(k, carry):
        for j in range(UNROLL):
            r = j
            u = j % NRB
            rm2 = (j - 2) % NIQ
            um2 = (j - 2) % NRB
            t = k * UNROLL + j

            @pl.when(t >= 2)
            def _():
                wait_scatter(rm2, um2)

            @pl.when(t + 6 < NCH)
            def _():
                fire_idx(t + 6, (j + 6) % NIQ)

            wait_gather(r, u)
            fire_scatter(r, u)

            @pl.when(t + 2 < NCH)
            def _():
                wait_idx((j + 2) % NIQ)
                fire_gather((j + 2) % NIQ, (j + 2) % NRB)
        return carry

    lax.fori_loop(0, MAIN // UNROLL, outer, 0)

    for t in range(MAIN, NCH):
        wait_scatter((t - 2) % NIQ, (t - 2) % NRB)
        if t + 6 < NCH:
            fire_idx(t + 6, (t + 6) % NIQ)
        wait_gather(t % NIQ, t % NRB)
        fire_scatter(t % NIQ, t % NRB)
        if t + 2 < NCH:
            wait_idx((t + 2) % NIQ)
            fire_gather((t + 2) % NIQ, (t + 2) % NRB)
    for t in range(NCH - 2, NCH):
        wait_scatter(t % NIQ, t % NRB)

    plsc.subcore_barrier()
    for k in range(ROWS_PER_TILE // CHUNK):
        u = k % 2
        ksl = pl.ds(row0 + k * CHUNK, CHUNK)
        if k >= 2:
            pltpu.make_async_copy(rbs[u], out_s_hbm.at[cid, ksl],
                                  ss[u]).wait()
        pltpu.sync_copy(acc_sh.at[ksl], rbs[u])
        pltpu.async_copy(rbs[u], out_s_hbm.at[cid, ksl], ss[u])
    for u in range(2):
        pltpu.make_async_copy(rbs[u], out_s_hbm.at[cid, pl.ds(row0, CHUNK)],
                              ss[u]).wait()
    pltpu.sync_copy(deg_sh.at[rsl], out_d_hbm.at[cid, rsl])


@jax.jit
def _sc_aggregate(h, src, dst, ones, z128, z1):
    mesh = plsc.VectorSubcoreMesh(core_axis_name="c", subcore_axis_name="s")
    return pl.kernel(
        _sc_aggregate_body,
        out_type=[
            jax.ShapeDtypeStruct((NC, N_PAD, D_IN), jnp.float32),
            jax.ShapeDtypeStruct((NC, N_PAD), jnp.float32),
        ],
        mesh=mesh,
        scratch_types=(
            [pltpu.VMEM((CHUNK,), jnp.int32)] * (2 * NIQ)
            + [pltpu.VMEM((CHUNK, D_IN), jnp.float32)] * NRB
            + [
                pltpu.VMEM((CHUNK,), jnp.float32),
                pltpu.VMEM_SHARED((N_PAD, D_IN), jnp.float32),
                pltpu.VMEM_SHARED((N_PAD,), jnp.float32),
            ]
            + [pltpu.SemaphoreType.DMA] * (NIQ + 2 * NRB + 1)
        ),
    )(h, src, dst, ones, z128, z1)


_TC_R = 1024


def _tc_combine_body(h_ref, s_ref, d_ref, wt_ref, b_ref, out_ref):
    s = s_ref[0] + s_ref[1]
    dg = d_ref[0] + d_ref[1]
    rb = _TC_R // 128
    ri = lax.broadcasted_iota(jnp.int32, (_TC_R, rb), 0) // 128
    ci = lax.broadcasted_iota(jnp.int32, (_TC_R, rb), 1)
    sel = (ri == ci).astype(jnp.float32)
    t = jnp.dot(sel, dg, preferred_element_type=jnp.float32)
    ji = lax.broadcasted_iota(jnp.int32, (_TC_R, 128), 0) % 128
    ki = lax.broadcasted_iota(jnp.int32, (_TC_R, 128), 1)
    deg_col = jnp.sum(jnp.where(ji == ki, t, 0.0), axis=1, keepdims=True)
    h_n = s / jnp.maximum(deg_col, 1.0)
    out_ref[...] = (
        jnp.dot(h_ref[...], wt_ref[0:D_IN, :],
                preferred_element_type=jnp.float32)
        + jnp.dot(h_n, wt_ref[D_IN:, :], preferred_element_type=jnp.float32)
        + b_ref[...]
    )


@jax.jit
def _tc_combine(h, s_parts, d_parts, wt, b2d):
    grid = (N_PAD // _TC_R,)
    rb = _TC_R // 128
    return pl.pallas_call(
        _tc_combine_body,
        grid=grid,
        in_specs=[
            pl.BlockSpec((_TC_R, D_IN), lambda i: (i, 0)),
            pl.BlockSpec((NC, _TC_R, D_IN), lambda i: (0, i, 0)),
            pl.BlockSpec((NC, rb, 128), lambda i: (0, i, 0)),
            pl.BlockSpec((2 * D_IN, D_OUT), lambda i: (0, 0)),
            pl.BlockSpec((1, D_OUT), lambda i: (0, 0)),
        ],
        out_specs=pl.BlockSpec((_TC_R, D_OUT), lambda i: (i, 0)),
        out_shape=jax.ShapeDtypeStruct((N_PAD, D_OUT), jnp.float32),
    )(h, s_parts, d_parts, wt, b2d)


def kernel(h, edge_index, W, b):
    ei = edge_index.astype(jnp.int32)
    src = ei[0]
    dst = ei[1]
    ones = jnp.ones((CHUNK,), jnp.float32)
    z128 = jnp.zeros((CHUNK, D_IN), jnp.float32)
    z1 = jnp.zeros((N_PAD,), jnp.float32)
    wt = W.T
    b2d = b.reshape(1, D_OUT)
    s_parts, d_parts = _sc_aggregate(h, src, dst, ones, z128, z1)
    d_parts = d_parts.reshape(NC, N_PAD // 128, 128)
    out = _tc_combine(h, s_parts, d_parts, wt, b2d)
    return out[:N_NODES]


# speedup vs baseline: 1.0472x; 1.0004x over previous
import jax
import jax.numpy as jnp
from jax import lax
from jax.experimental import pallas as pl
from jax.experimental.pallas import tpu as pltpu
from jax.experimental.pallas import tpu_sc as plsc

N_NODES = 10000
N_PAD = 10240
N_EDGES = 320000
D_IN = 128
D_OUT = 128

NC = 2
NS = 16
NW = NC * NS

EPW = N_EDGES // NW
CHUNK = 80
NCH = EPW // CHUNK
ROWS_PER_TILE = N_PAD // NS


NIQ = 8
NRB = 4


def _sc_aggregate_body(h_hbm, src_hbm, dst_hbm, ones_hbm, z128_hbm, z1_hbm,
                       out_s_hbm, out_d_hbm, *refs):
    sis = list(refs[0:NIQ])
    dis = list(refs[NIQ:2 * NIQ])
    rbs = list(refs[2 * NIQ:2 * NIQ + NRB])
    ones_v = refs[2 * NIQ + NRB]
    dummy2c = refs[2 * NIQ + NRB + 1]
    acc_sh = refs[2 * NIQ + NRB + 2]
    deg_sh = refs[2 * NIQ + NRB + 3]
    sems = refs[2 * NIQ + NRB + 4:]
    gis = list(sems[0:NIQ])
    gs = list(sems[NIQ:NIQ + NRB])
    ss = list(sems[NIQ + NRB:NIQ + 2 * NRB])
    dsem = sems[NIQ + 2 * NRB]
    cid = lax.axis_index("c")
    sid = lax.axis_index("s")
    wid = cid * NS + sid

    row0 = pl.multiple_of(sid * ROWS_PER_TILE, 8)
    rsl = pl.ds(row0, ROWS_PER_TILE)
    ebase = wid * EPW

    def esl(t):
        return pl.ds(pl.multiple_of(ebase + t * CHUNK, 8), CHUNK)

    def fire_idx(t, q):
        pltpu.async_copy(src_hbm.at[esl(t)], sis[q], gis[q])
        pltpu.async_copy(dst_hbm.at[esl(t)], dis[q], gis[q])

    def wait_idx(q):
        pltpu.make_async_copy(src_hbm.at[pl.ds(0, 2 * CHUNK)], dummy2c,
                              gis[q]).wait()

    def fire_gather(q, u):
        pltpu.async_copy(h_hbm.at[sis[q]], rbs[u], gs[u])

    def wait_gather(q, u):
        pltpu.make_async_copy(h_hbm.at[sis[q]], rbs[u], gs[u]).wait()

    def fire_scatter(q, u):
        pltpu.async_copy(ones_v, deg_sh.at[dis[q]], dsem, add=True)
        pltpu.async_copy(rbs[u], acc_sh.at[dis[q]], ss[u], add=True)

    def wait_scatter(q, u):
        pltpu.make_async_copy(rbs[u], acc_sh.at[dis[q]], ss[u]).wait()
        pltpu.make_async_copy(ones_v, deg_sh.at[dis[q]], dsem).wait()

    for q in range(6):
        fire_idx(q, q)
    pltpu.sync_copy(z128_hbm, rbs[0])
    for k in range(ROWS_PER_TILE // CHUNK):
        pltpu.async_copy(rbs[0], acc_sh.at[pl.ds(row0 + k * CHUNK, CHUNK)],
                         ss[k % NRB])
    pltpu.sync_copy(z1_hbm.at[rsl], deg_sh.at[rsl])
    pltpu.sync_copy(ones_hbm, ones_v)
    for k in range(ROWS_PER_TILE // CHUNK):
        pltpu.make_async_copy(rbs[0],
                              acc_sh.at[pl.ds(row0 + k * CHUNK, CHUNK)],
                              ss[k % NRB]).wait()
    for u in range(2):
        wait_idx(u)
        fire_gather(u, u)
    plsc.subcore_barrier()

    UNROLL = 8
    MAIN = (NCH - 5) // UNROLL * UNROLL

    def outer(k, carry):
        for j in range(UNROLL):
            r = j
            u = j % NRB
            rm2 = (j - 2) % NIQ
            um2 = (j - 2) % NRB
            t = k * UNROLL + j

            @pl.when(t >= 2)
            def _():
                wait_scatter(rm2, um2)

            @pl.when(t + 6 < NCH)
            def _():
                fire_idx(t + 6, (j + 6) % NIQ)

            wait_gather(r, u)
            fire_scatter(r, u)

            @pl.when(t + 2 < NCH)
            def _():
                wait_idx((j + 2) % NIQ)
                fire_gather((j + 2) % NIQ, (j + 2) % NRB)
        return carry

    lax.fori_loop(0, MAIN // UNROLL, outer, 0)

    for t in range(MAIN, NCH):
        wait_scatter((t - 2) % NIQ, (t - 2) % NRB)
        if t + 6 < NCH:
            fire_idx(t + 6, (t + 6) % NIQ)
        wait_gather(t % NIQ, t % NRB)
        fire_scatter(t % NIQ, t % NRB)
        if t + 2 < NCH:
            wait_idx((t + 2) % NIQ)
            fire_gather((t + 2) % NIQ, (t + 2) % NRB)
    for t in range(NCH - 2, NCH):
        wait_scatter(t % NIQ, t % NRB)

    plsc.subcore_barrier()
    for k in range(ROWS_PER_TILE // CHUNK):
        u = k % 2
        ksl = pl.ds(row0 + k * CHUNK, CHUNK)
        if k >= 2:
            pltpu.make_async_copy(rbs[u], out_s_hbm.at[cid, ksl],
                                  ss[u]).wait()
        pltpu.sync_copy(acc_sh.at[ksl], rbs[u])
        pltpu.async_copy(rbs[u], out_s_hbm.at[cid, ksl], ss[u])
    for u in range(2):
        pltpu.make_async_copy(rbs[u], out_s_hbm.at[cid, pl.ds(row0, CHUNK)],
                              ss[u]).wait()
    pltpu.sync_copy(deg_sh.at[rsl], out_d_hbm.at[cid, rsl])


@jax.jit
def _sc_aggregate(h, src, dst, ones, z128, z1):
    mesh = plsc.VectorSubcoreMesh(core_axis_name="c", subcore_axis_name="s")
    return pl.kernel(
        _sc_aggregate_body,
        out_type=[
            jax.ShapeDtypeStruct((NC, N_PAD, D_IN), jnp.float32),
            jax.ShapeDtypeStruct((NC, N_PAD), jnp.float32),
        ],
        mesh=mesh,
        scratch_types=(
            [pltpu.VMEM((CHUNK,), jnp.int32)] * (2 * NIQ)
            + [pltpu.VMEM((CHUNK, D_IN), jnp.float32)] * NRB
            + [
                pltpu.VMEM((CHUNK,), jnp.float32),
                pltpu.VMEM((2 * CHUNK,), jnp.int32),
                pltpu.VMEM_SHARED((N_PAD, D_IN), jnp.float32),
                pltpu.VMEM_SHARED((N_PAD,), jnp.float32),
            ]
            + [pltpu.SemaphoreType.DMA] * (NIQ + 2 * NRB + 1)
        ),
    )(h, src, dst, ones, z128, z1)


_TC_R = 1024


def _tc_combine_body(h_ref, s_ref, d_ref, wt_ref, b_ref, out_ref):
    s = s_ref[0] + s_ref[1]
    dg = d_ref[0] + d_ref[1]
    rb = _TC_R // 128
    ri = lax.broadcasted_iota(jnp.int32, (_TC_R, rb), 0) // 128
    ci = lax.broadcasted_iota(jnp.int32, (_TC_R, rb), 1)
    sel = (ri == ci).astype(jnp.float32)
    t = jnp.dot(sel, dg, preferred_element_type=jnp.float32)
    ji = lax.broadcasted_iota(jnp.int32, (_TC_R, 128), 0) % 128
    ki = lax.broadcasted_iota(jnp.int32, (_TC_R, 128), 1)
    deg_col = jnp.sum(jnp.where(ji == ki, t, 0.0), axis=1, keepdims=True)
    h_n = s / jnp.maximum(deg_col, 1.0)
    out_ref[...] = (
        jnp.dot(h_ref[...], wt_ref[0:D_IN, :],
                preferred_element_type=jnp.float32)
        + jnp.dot(h_n, wt_ref[D_IN:, :], preferred_element_type=jnp.float32)
        + b_ref[...]
    )


@jax.jit
def _tc_combine(h, s_parts, d_parts, wt, b2d):
    grid = (N_PAD // _TC_R,)
    rb = _TC_R // 128
    return pl.pallas_call(
        _tc_combine_body,
        grid=grid,
        in_specs=[
            pl.BlockSpec((_TC_R, D_IN), lambda i: (i, 0)),
            pl.BlockSpec((NC, _TC_R, D_IN), lambda i: (0, i, 0)),
            pl.BlockSpec((NC, rb, 128), lambda i: (0, i, 0)),
            pl.BlockSpec((2 * D_IN, D_OUT), lambda i: (0, 0)),
            pl.BlockSpec((1, D_OUT), lambda i: (0, 0)),
        ],
        out_specs=pl.BlockSpec((_TC_R, D_OUT), lambda i: (i, 0)),
        out_shape=jax.ShapeDtypeStruct((N_PAD, D_OUT), jnp.float32),
    )(h, s_parts, d_parts, wt, b2d)


def kernel(h, edge_index, W, b):
    ei = edge_index.astype(jnp.int32)
    src = ei[0]
    dst = ei[1]
    ones = jnp.ones((CHUNK,), jnp.float32)
    z128 = jnp.zeros((CHUNK, D_IN), jnp.float32)
    z1 = jnp.zeros((N_PAD,), jnp.float32)
    wt = W.T
    b2d = b.reshape(1, D_OUT)
    s_parts, d_parts = _sc_aggregate(h, src, dst, ones, z128, z1)
    d_parts = d_parts.reshape(NC, N_PAD // 128, 128)
    out = _tc_combine(h, s_parts, d_parts, wt, b2d)
    return out[:N_NODES]


# speedup vs baseline: 1.0476x; 1.0003x over previous
import jax
import jax.numpy as jnp
from jax import lax
from jax.experimental import pallas as pl
from jax.experimental.pallas import tpu as pltpu
from jax.experimental.pallas import tpu_sc as plsc

N_NODES = 10000
N_PAD = 10240
N_EDGES = 320000
D_IN = 128
D_OUT = 128

NC = 2
NS = 16
NW = NC * NS

EPW = N_EDGES // NW
CHUNK = 80
NCH = EPW // CHUNK
ROWS_PER_TILE = N_PAD // NS


NIQ = 8
NRB = 4


def _sc_aggregate_body(h_hbm, src_hbm, dst_hbm, ones_hbm, z128_hbm, z1_hbm,
                       out_s_hbm, out_d_hbm, *refs):
    sis = list(refs[0:NIQ])
    dis = list(refs[NIQ:2 * NIQ])
    rbs = list(refs[2 * NIQ:2 * NIQ + NRB])
    ones_v = refs[2 * NIQ + NRB]
    dummy2c = refs[2 * NIQ + NRB + 1]
    acc_sh = refs[2 * NIQ + NRB + 2]
    deg_sh = refs[2 * NIQ + NRB + 3]
    sems = refs[2 * NIQ + NRB + 4:]
    gis = list(sems[0:NIQ])
    gs = list(sems[NIQ:NIQ + NRB])
    ss = list(sems[NIQ + NRB:NIQ + 2 * NRB])
    dsem = sems[NIQ + 2 * NRB]
    cid = lax.axis_index("c")
    sid = lax.axis_index("s")
    wid = cid * NS + sid

    row0 = pl.multiple_of(sid * ROWS_PER_TILE, 8)
    rsl = pl.ds(row0, ROWS_PER_TILE)
    ebase = wid * EPW

    def esl(t):
        return pl.ds(pl.multiple_of(ebase + t * CHUNK, 8), CHUNK)

    def fire_idx(t, q):
        pltpu.async_copy(src_hbm.at[esl(t)], sis[q], gis[q])
        pltpu.async_copy(dst_hbm.at[esl(t)], dis[q], gis[q])

    def wait_idx(q):
        pltpu.make_async_copy(src_hbm.at[pl.ds(0, 2 * CHUNK)], dummy2c,
                              gis[q]).wait()

    def fire_gather(q, u):
        pltpu.async_copy(h_hbm.at[sis[q]], rbs[u], gs[u])

    def wait_gather(q, u):
        pltpu.make_async_copy(h_hbm.at[sis[q]], rbs[u], gs[u]).wait()

    def fire_scatter(q, u):
        pltpu.async_copy(ones_v, deg_sh.at[dis[q]], dsem, add=True)
        pltpu.async_copy(rbs[u], acc_sh.at[dis[q]], ss[u], add=True)

    def wait_scatter(q, u):
        pltpu.make_async_copy(rbs[u], acc_sh.at[dis[q]], ss[u]).wait()
        pltpu.make_async_copy(ones_v, deg_sh.at[dis[q]], dsem).wait()

    for q in range(6):
        fire_idx(q, q)
    pltpu.sync_copy(z128_hbm, rbs[0])
    for k in range(ROWS_PER_TILE // CHUNK):
        pltpu.async_copy(rbs[0], acc_sh.at[pl.ds(row0 + k * CHUNK, CHUNK)],
                         ss[k % NRB])
    pltpu.sync_copy(z1_hbm.at[rsl], deg_sh.at[rsl])
    pltpu.sync_copy(ones_hbm, ones_v)
    for k in range(ROWS_PER_TILE // CHUNK):
        pltpu.make_async_copy(rbs[0],
                              acc_sh.at[pl.ds(row0 + k * CHUNK, CHUNK)],
                              ss[k % NRB]).wait()
    for u in range(2):
        wait_idx(u)
        fire_gather(u, u)
    plsc.subcore_barrier()

    UNROLL = 8
    MAIN = (NCH - 5) // UNROLL * UNROLL

    def outer(k, carry):
        for j in range(UNROLL):
            r = j
            u = j % NRB
            rm2 = (j - 2) % NIQ
            um2 = (j - 2) % NRB
            t = k * UNROLL + j

            @pl.when(t >= 2)
            def _():
                wait_scatter(rm2, um2)

            @pl.when(t + 6 < NCH)
            def _():
                fire_idx(t + 6, (j + 6) % NIQ)

            wait_gather(r, u)
            fire_scatter(r, u)

            @pl.when(t + 2 < NCH)
            def _():
                wait_idx((j + 2) % NIQ)
                fire_gather((j + 2) % NIQ, (j + 2) % NRB)
        return carry

    lax.fori_loop(0, MAIN // UNROLL, outer, 0)

    for t in range(MAIN, NCH):
        wait_scatter((t - 2) % NIQ, (t - 2) % NRB)
        if t + 6 < NCH:
            fire_idx(t + 6, (t + 6) % NIQ)
        wait_gather(t % NIQ, t % NRB)
        fire_scatter(t % NIQ, t % NRB)
        if t + 2 < NCH:
            wait_idx((t + 2) % NIQ)
            fire_gather((t + 2) % NIQ, (t + 2) % NRB)
    for t in range(NCH - 2, NCH):
        wait_scatter(t % NIQ, t % NRB)

    plsc.subcore_barrier()
    for k in range(ROWS_PER_TILE // CHUNK):
        u = k % NRB
        ksl = pl.ds(row0 + k * CHUNK, CHUNK)
        if k >= NRB:
            pltpu.make_async_copy(rbs[u], out_s_hbm.at[cid, ksl],
                                  ss[u]).wait()
        pltpu.sync_copy(acc_sh.at[ksl], rbs[u])
        pltpu.async_copy(rbs[u], out_s_hbm.at[cid, ksl], ss[u])
    for u in range(NRB):
        pltpu.make_async_copy(rbs[u], out_s_hbm.at[cid, pl.ds(row0, CHUNK)],
                              ss[u]).wait()
    pltpu.sync_copy(deg_sh.at[rsl], out_d_hbm.at[cid, rsl])


@jax.jit
def _sc_aggregate(h, src, dst, ones, z128, z1):
    mesh = plsc.VectorSubcoreMesh(core_axis_name="c", subcore_axis_name="s")
    return pl.kernel(
        _sc_aggregate_body,
        out_type=[
            jax.ShapeDtypeStruct((NC, N_PAD, D_IN), jnp.float32),
            jax.ShapeDtypeStruct((NC, N_PAD), jnp.float32),
        ],
        mesh=mesh,
        scratch_types=(
            [pltpu.VMEM((CHUNK,), jnp.int32)] * (2 * NIQ)
            + [pltpu.VMEM((CHUNK, D_IN), jnp.float32)] * NRB
            + [
                pltpu.VMEM((CHUNK,), jnp.float32),
                pltpu.VMEM((2 * CHUNK,), jnp.int32),
                pltpu.VMEM_SHARED((N_PAD, D_IN), jnp.float32),
                pltpu.VMEM_SHARED((N_PAD,), jnp.float32),
            ]
            + [pltpu.SemaphoreType.DMA] * (NIQ + 2 * NRB + 1)
        ),
    )(h, src, dst, ones, z128, z1)


_TC_R = 1024


def _tc_combine_body(h_ref, s_ref, d_ref, wt_ref, b_ref, out_ref):
    s = s_ref[0] + s_ref[1]
    dg = d_ref[0] + d_ref[1]
    rb = _TC_R // 128
    ri = lax.broadcasted_iota(jnp.int32, (_TC_R, rb), 0) // 128
    ci = lax.broadcasted_iota(jnp.int32, (_TC_R, rb), 1)
    sel = (ri == ci).astype(jnp.float32)
    t = jnp.dot(sel, dg, preferred_element_type=jnp.float32)
    ji = lax.broadcasted_iota(jnp.int32, (_TC_R, 128), 0) % 128
    ki = lax.broadcasted_iota(jnp.int32, (_TC_R, 128), 1)
    deg_col = jnp.sum(jnp.where(ji == ki, t, 0.0), axis=1, keepdims=True)
    h_n = s / jnp.maximum(deg_col, 1.0)
    out_ref[...] = (
        jnp.dot(h_ref[...], wt_ref[0:D_IN, :],
                preferred_element_type=jnp.float32)
        + jnp.dot(h_n, wt_ref[D_IN:, :], preferred_element_type=jnp.float32)
        + b_ref[...]
    )


@jax.jit
def _tc_combine(h, s_parts, d_parts, wt, b2d):
    grid = (N_PAD // _TC_R,)
    rb = _TC_R // 128
    return pl.pallas_call(
        _tc_combine_body,
        grid=grid,
        in_specs=[
            pl.BlockSpec((_TC_R, D_IN), lambda i: (i, 0)),
            pl.BlockSpec((NC, _TC_R, D_IN), lambda i: (0, i, 0)),
            pl.BlockSpec((NC, rb, 128), lambda i: (0, i, 0)),
            pl.BlockSpec((2 * D_IN, D_OUT), lambda i: (0, 0)),
            pl.BlockSpec((1, D_OUT), lambda i: (0, 0)),
        ],
        out_specs=pl.BlockSpec((_TC_R, D_OUT), lambda i: (i, 0)),
        out_shape=jax.ShapeDtypeStruct((N_PAD, D_OUT), jnp.float32),
    )(h, s_parts, d_parts, wt, b2d)


def kernel(h, edge_index, W, b):
    ei = edge_index.astype(jnp.int32)
    src = ei[0]
    dst = ei[1]
    ones = jnp.ones((CHUNK,), jnp.float32)
    z128 = jnp.zeros((CHUNK, D_IN), jnp.float32)
    z1 = jnp.zeros((N_PAD,), jnp.float32)
    wt = W.T
    b2d = b.reshape(1, D_OUT)
    s_parts, d_parts = _sc_aggregate(h, src, dst, ones, z128, z1)
    d_parts = d_parts.reshape(NC, N_PAD // 128, 128)
    out = _tc_combine(h, s_parts, d_parts, wt, b2d)
    return out[:N_NODES]
